# baseline (speedup 1.0000x reference)
import functools

import jax
import jax.numpy as jnp
from jax import lax
from jax.experimental import pallas as pl
from jax.experimental.pallas import tpu as pltpu
from jax.experimental.pallas import tpu_sc as plsc

N = 100000
D = 128
V = 21
L = 16
C = 128
NW = 32
N_FULL = N // C
TAIL = N - N_FULL * C
CPW = (N_FULL + NW - 1) // NW
LAST_CH = N_FULL - (NW - 1) * CPW
BPW = CPW * C
LAST_IDS = LAST_CH * C + TAIL
NB = 4
RU = 16
STEPS = CPW + NB
OUTER = (STEPS + NB - 1) // NB


def _make_kernel():
  mesh = plsc.VectorSubcoreMesh(core_axis_name="c", subcore_axis_name="s")

  @functools.partial(
      pl.kernel,
      out_type=jax.ShapeDtypeStruct((N, D), jnp.float32),
      mesh=mesh,
      scratch_types=[
          pltpu.VMEM((BPW,), jnp.int32),
          pltpu.VMEM((V, D), jnp.float32),
          pltpu.VMEM((NB, C, D), jnp.float32),
          pltpu.VMEM((TAIL, D), jnp.float32),
          pltpu.SemaphoreType.DMA((NB,)),
      ],
  )
  def k(x_hbm, table_hbm, out_hbm, idx_v, table_v, rows, rows_t, sem_w):
    wid = lax.axis_index("s") * 2 + lax.axis_index("c")
    base_ch = wid * CPW
    n_my = jnp.where(wid == NW - 1, LAST_CH, CPW)

    pltpu.sync_copy(table_hbm, table_v)

    @pl.when(wid < NW - 1)
    def _():
      pltpu.sync_copy(x_hbm.at[pl.ds(wid * BPW, BPW)], idx_v)

    @pl.when(wid == NW - 1)
    def _():
      pltpu.sync_copy(x_hbm.at[pl.ds((NW - 1) * BPW, LAST_IDS)],
                      idx_v.at[pl.ds(0, LAST_IDS)])

    def copy_row(src_row, dst_ref, dst_row):
      for c in range(D // L):
        dst_ref[dst_row, pl.ds(c * L, L)] = table_v[src_row, pl.ds(c * L, L)]

    def build_chunk(j, buf):
      def grp(g, carry):
        r0 = j * C + g * RU
        iv = idx_v[pl.ds(r0, RU)]
        for u in range(RU):
          copy_row(iv[u], buf, g * RU + u)
        return carry
      lax.fori_loop(0, C // RU, grp, 0)

    def start_write(j, b):
      pltpu.async_copy(rows.at[b], out_hbm.at[pl.ds((base_ch + j) * C, C)],
                       sem_w.at[b])

    def wait_write(b):
      pltpu.make_async_copy(rows.at[b], out_hbm.at[pl.ds(0, C)],
                            sem_w.at[b]).wait()

    def outer_body(jj, carry):
      for b in range(NB):
        j = jj * NB + b

        @pl.when((j >= NB) & (j < n_my + NB))
        def _(b=b):
          wait_write(b)

        @pl.when(j < n_my)
        def _(j=j, b=b):
          build_chunk(j, rows.at[b])
          start_write(j, b)
      return carry

    lax.fori_loop(0, OUTER, outer_body, 0)

    @pl.when(wid == NW - 1)
    def _():
      def tgrp(g, carry):
        r0 = LAST_CH * C + g * RU
        iv = idx_v[pl.ds(r0, RU)]
        for u in range(RU):
          copy_row(iv[u], rows_t, g * RU + u)
        return carry
      lax.fori_loop(0, TAIL // RU, tgrp, 0)
      pltpu.sync_copy(rows_t, out_hbm.at[pl.ds(N_FULL * C, TAIL)])

  return k


_lookup = _make_kernel()


def kernel(x, table):
  return _lookup(x.astype(jnp.int32), table)


# speedup vs baseline: 2.9549x; 2.9549x over previous
import functools

import jax
import jax.numpy as jnp
from jax import lax
from jax.experimental import pallas as pl
from jax.experimental.pallas import tpu as pltpu
from jax.experimental.pallas import tpu_sc as plsc

N = 100000
D = 128
V = 21
C = 128
NW = 32
N_FULL = N // C
TAIL = N - N_FULL * C
CPW = (N_FULL + NW - 1) // NW
LAST_CH = N_FULL - (NW - 1) * CPW
BPW = CPW * C
LAST_IDS = LAST_CH * C + TAIL
NB = 4
PF = 2
STEPS = CPW + PF
OUTER = (STEPS + NB - 1) // NB


def _make_kernel():
  mesh = plsc.VectorSubcoreMesh(core_axis_name="c", subcore_axis_name="s")

  @functools.partial(
      pl.kernel,
      out_type=jax.ShapeDtypeStruct((N, D), jnp.float32),
      mesh=mesh,
      scratch_types=[
          pltpu.VMEM((BPW,), jnp.int32),
          pltpu.VMEM((NB, C, D), jnp.float32),
          pltpu.VMEM((TAIL, D), jnp.float32),
          pltpu.SemaphoreType.DMA((NB,)),
          pltpu.SemaphoreType.DMA((NB,)),
          pltpu.SemaphoreType.DMA,
          pltpu.VMEM_SHARED((V, D), jnp.float32),
      ],
  )
  def k(x_hbm, table_hbm, out_hbm,
        idx_v, rows, rows_t, sem_g, sem_w, sem_t, table_s):
    wid = lax.axis_index("s") * 2 + lax.axis_index("c")
    base_ch = wid * CPW
    n_my = jnp.where(wid == NW - 1, LAST_CH, CPW)

    @pl.when(lax.axis_index("s") == 0)
    def _():
      pltpu.sync_copy(table_hbm, table_s)

    @pl.when(wid < NW - 1)
    def _():
      pltpu.sync_copy(x_hbm.at[pl.ds(wid * BPW, BPW)], idx_v)

    @pl.when(wid == NW - 1)
    def _():
      pltpu.sync_copy(x_hbm.at[pl.ds((NW - 1) * BPW, LAST_IDS)],
                      idx_v.at[pl.ds(0, LAST_IDS)])

    plsc.subcore_barrier()

    def start_gather(j, b):
      pltpu.async_copy(table_s.at[idx_v.at[pl.ds(j * C, C)]], rows.at[b],
                       sem_g.at[b])

    def start_write(j, b):
      pltpu.async_copy(rows.at[b], out_hbm.at[pl.ds((base_ch + j) * C, C)],
                       sem_w.at[b])

    def wait_gather(b):
      pltpu.make_async_copy(out_hbm.at[pl.ds(0, C)], rows.at[b],
                            sem_g.at[b]).wait()

    def wait_write(b):
      pltpu.make_async_copy(rows.at[b], out_hbm.at[pl.ds(0, C)],
                            sem_w.at[b]).wait()

    for b in range(PF):
      @pl.when(b < n_my)
      def _(b=b):
        start_gather(jnp.int32(b), b)

    def outer_body(jj, carry):
      for b in range(NB):
        j = jj * NB + b
        b2 = (b + PF) % NB

        @pl.when((j >= PF) & (j < n_my + PF))
        def _(b2=b2):
          wait_write(b2)

        @pl.when(j + PF < n_my)
        def _(j=j, b2=b2):
          start_gather(j + PF, b2)

        @pl.when(j < n_my)
        def _(j=j, b=b):
          wait_gather(b)
          start_write(j, b)
      return carry

    lax.fori_loop(0, OUTER, outer_body, 0)

    @pl.when(wid == NW - 1)
    def _():
      pltpu.async_copy(table_s.at[idx_v.at[pl.ds(LAST_CH * C, TAIL)]],
                       rows_t, sem_t).wait()
      pltpu.sync_copy(rows_t, out_hbm.at[pl.ds(N_FULL * C, TAIL)])

  return k


_lookup = _make_kernel()


def kernel(x, table):
  return _lookup(x.astype(jnp.int32), table)
